# baseline (speedup 1.0000x reference)
import functools

import jax
import jax.numpy as jnp
from jax import lax
from jax.experimental import pallas as pl
from jax.experimental.pallas import tpu as pltpu
from jax.experimental.pallas import tpu_sc as plsc

VOCAB = 1000000
D = 64
O = 2
B = 4096
S = 200

NT = 32
BPT = B // NT
TPT = S * BPT
CH = 128
NCH = TPT // CH


VH = VOCAB // 2


def _tw_body(w_ref, t_ref, e0_ref, e1_ref, q0_ref, q1_ref):
    res = lax.dot_general(w_ref[...], t_ref[...], (((0,), (1,)), ((), ())),
                          preferred_element_type=jnp.float32)
    e0_ref[...] = res[0, :]
    e1_ref[...] = res[1, :]
    q0_ref[...] = res[2, :]
    q1_ref[...] = res[3, :]


def _table_w(table2, W2):
    BLK = 8192
    oshape = jax.ShapeDtypeStruct((VH,), jnp.float32)
    ospec = pl.BlockSpec((BLK,), lambda i: (i,))
    return pl.pallas_call(
        _tw_body,
        grid=(pl.cdiv(VH, BLK),),
        in_specs=[
            pl.BlockSpec((2 * D, 2 * O), lambda i: (0, 0)),
            pl.BlockSpec((BLK, 2 * D), lambda i: (i, 0)),
        ],
        out_specs=[ospec, ospec, ospec, ospec],
        out_shape=[oshape, oshape, oshape, oshape],
    )(W2, table2)


def _xp_body(t_ref, o_ref):
    a = t_ref[...].T
    o_ref[...] = (a >> 1) + (a & 1) * VH


def _transpose_text(text):
    return pl.pallas_call(
        _xp_body,
        grid=(NT,),
        in_specs=[pl.BlockSpec((BPT, S), lambda i: (i, 0))],
        out_specs=pl.BlockSpec((S, BPT), lambda i: (0, i)),
        out_shape=jax.ShapeDtypeStruct((S, B), jnp.int32),
    )(text)


@functools.partial(
    pl.kernel,
    out_type=[
        jax.ShapeDtypeStruct((B,), jnp.float32),
        jax.ShapeDtypeStruct((B,), jnp.float32),
    ],
    mesh=plsc.VectorSubcoreMesh(core_axis_name="c", subcore_axis_name="s"),
    scratch_types=[
        pltpu.VMEM((NCH, CH), jnp.int32),
        pltpu.VMEM((TPT,), jnp.float32),
        pltpu.VMEM((TPT,), jnp.float32),
        pltpu.VMEM((BPT,), jnp.float32),
        pltpu.VMEM((32,), jnp.float32),
        pltpu.VMEM((BPT,), jnp.float32),
        pltpu.VMEM((BPT,), jnp.float32),
        pltpu.SemaphoreType.DMA,
    ],
)
def _sc_pool(textt_hbm, len_hbm, b16_hbm, tw0_hbm, tw1_hbm,
             out0_hbm, out1_hbm,
             text_v, r0_v, r1_v, len_v, b_v, o0_v, o1_v, sem):
    wid = lax.axis_index("s") * 2 + lax.axis_index("c")
    bbase = wid * BPT

    pltpu.sync_copy(textt_hbm.at[:, pl.ds(bbase, BPT)], text_v)
    pltpu.sync_copy(len_hbm.at[pl.ds(bbase, BPT)], len_v)
    pltpu.sync_copy(b16_hbm, b_v)

    def chunk_copies(j):
        c0 = pltpu.make_async_copy(tw0_hbm.at[text_v.at[j]],
                                   r0_v.at[pl.ds(j * CH, CH)], sem)
        c1 = pltpu.make_async_copy(tw1_hbm.at[text_v.at[j]],
                                   r1_v.at[pl.ds(j * CH, CH)], sem)
        return c0, c1

    def fire(j, carry):
        c0, c1 = chunk_copies(j)
        c0.start()
        c1.start()

        @pl.when(j > 0)
        def _():
            p0, p1 = chunk_copies(j - 1)
            p0.wait()
            p1.wait()

        return carry

    lax.fori_loop(0, NCH, fire, 0)
    l0, l1 = chunk_copies(NCH - 1)
    l0.wait()
    l1.wait()

    b0vec = b_v[pl.ds(0, 16)]
    b1vec = b_v[pl.ds(16, 16)]
    zero = jnp.zeros((16,), jnp.float32)
    for c in range(BPT // 16):
        coff = c * 16

        def tb(k, accs, coff=coff):
            a0, a1 = accs
            for i in range(8):
                off = (k * 8 + i) * CH + coff
                a0 = a0 + r0_v[pl.ds(off, 16)]
                a1 = a1 + r1_v[pl.ds(off, 16)]
            return a0, a1

        a0, a1 = lax.fori_loop(0, S // 8, tb, (zero, zero))
        lenvec = len_v[pl.ds(coff, 16)]
        o0_v[pl.ds(coff, 16)] = a0 * lenvec + b0vec
        o1_v[pl.ds(coff, 16)] = a1 * lenvec + b1vec

    pltpu.sync_copy(o0_v, out0_hbm.at[pl.ds(bbase, BPT)])
    pltpu.sync_copy(o1_v, out1_hbm.at[pl.ds(bbase, BPT)])


def kernel(text, text_lengths, table, W, b):
    textt = _transpose_text(text.astype(jnp.int32))
    inv_len = 1.0 / text_lengths.astype(jnp.float32)
    b16 = jnp.repeat(b.astype(jnp.float32), 16)
    table2 = table.reshape(VH, 2 * D)
    Wf = W.astype(jnp.float32)
    W2 = jnp.zeros((2 * D, 2 * O), jnp.float32)
    W2 = W2.at[:D, :O].set(Wf).at[D:, O:].set(Wf)
    e0, e1, q0, q1 = _table_w(table2, W2)
    tw0 = jnp.concatenate([e0, q0])
    tw1 = jnp.concatenate([e1, q1])
    out0, out1 = _sc_pool(textt, inv_len, b16, tw0, tw1)
    return jnp.stack([out0, out1], axis=1)


# speedup vs baseline: 1.0317x; 1.0317x over previous
import functools

import jax
import jax.numpy as jnp
from jax import lax
from jax.experimental import pallas as pl
from jax.experimental.pallas import tpu as pltpu
from jax.experimental.pallas import tpu_sc as plsc

VOCAB = 1000000
D = 64
O = 2
B = 4096
S = 200

NT = 32
BPT = B // NT
TPT = S * BPT
CH = 128
NCH = TPT // CH


VH = VOCAB // 2


def _tw_body(w_ref, t_ref, e0_ref, e1_ref, q0_ref, q1_ref):
    res = lax.dot_general(w_ref[...], t_ref[...], (((0,), (1,)), ((), ())),
                          preferred_element_type=jnp.float32)
    e0_ref[...] = res[0, :]
    e1_ref[...] = res[1, :]
    q0_ref[...] = res[2, :]
    q1_ref[...] = res[3, :]


def _table_w(table2, W2):
    BLK = 8192
    oshape = jax.ShapeDtypeStruct((VH,), jnp.float32)
    ospec = pl.BlockSpec((BLK,), lambda i: (i,))
    return pl.pallas_call(
        _tw_body,
        grid=(pl.cdiv(VH, BLK),),
        in_specs=[
            pl.BlockSpec((2 * D, 2 * O), lambda i: (0, 0)),
            pl.BlockSpec((BLK, 2 * D), lambda i: (i, 0)),
        ],
        out_specs=[ospec, ospec, ospec, ospec],
        out_shape=[oshape, oshape, oshape, oshape],
    )(W2, table2)


def _xp_body(t_ref, o_ref):
    a = t_ref[...].T
    o_ref[...] = (a >> 1) + (a & 1) * VH


def _transpose_text(text):
    return pl.pallas_call(
        _xp_body,
        grid=(NT,),
        in_specs=[pl.BlockSpec((BPT, S), lambda i: (i, 0))],
        out_specs=pl.BlockSpec((S, BPT), lambda i: (i, 0)),
        out_shape=jax.ShapeDtypeStruct((NT * S, BPT), jnp.int32),
    )(text)


@functools.partial(
    pl.kernel,
    out_type=[
        jax.ShapeDtypeStruct((B,), jnp.float32),
        jax.ShapeDtypeStruct((B,), jnp.float32),
    ],
    mesh=plsc.VectorSubcoreMesh(core_axis_name="c", subcore_axis_name="s"),
    scratch_types=[
        pltpu.VMEM((NCH, CH), jnp.int32),
        pltpu.VMEM((TPT,), jnp.float32),
        pltpu.VMEM((TPT,), jnp.float32),
        pltpu.VMEM((BPT,), jnp.float32),
        pltpu.VMEM((32,), jnp.float32),
        pltpu.VMEM((BPT,), jnp.float32),
        pltpu.VMEM((BPT,), jnp.float32),
        pltpu.SemaphoreType.DMA,
    ],
)
def _sc_pool(textt_hbm, len_hbm, b16_hbm, tw0_hbm, tw1_hbm,
             out0_hbm, out1_hbm,
             text_v, r0_v, r1_v, len_v, b_v, o0_v, o1_v, sem):
    wid = lax.axis_index("s") * 2 + lax.axis_index("c")
    bbase = wid * BPT

    pltpu.sync_copy(textt_hbm.at[pl.ds(wid * S, S)], text_v)
    pltpu.sync_copy(len_hbm.at[pl.ds(bbase, BPT)], len_v)
    pltpu.sync_copy(b16_hbm, b_v)

    def chunk_copies(j):
        c0 = pltpu.make_async_copy(tw0_hbm.at[text_v.at[j]],
                                   r0_v.at[pl.ds(j * CH, CH)], sem)
        c1 = pltpu.make_async_copy(tw1_hbm.at[text_v.at[j]],
                                   r1_v.at[pl.ds(j * CH, CH)], sem)
        return c0, c1

    def fire(j, carry):
        c0, c1 = chunk_copies(j)
        c0.start()
        c1.start()

        @pl.when(j > 0)
        def _():
            p0, p1 = chunk_copies(j - 1)
            p0.wait()
            p1.wait()

        return carry

    lax.fori_loop(0, NCH, fire, 0)
    l0, l1 = chunk_copies(NCH - 1)
    l0.wait()
    l1.wait()

    b0vec = b_v[pl.ds(0, 16)]
    b1vec = b_v[pl.ds(16, 16)]
    zero = jnp.zeros((16,), jnp.float32)
    for c in range(BPT // 16):
        coff = c * 16

        def tb(k, accs, coff=coff):
            a0, a1 = accs
            for i in range(8):
                off = (k * 8 + i) * CH + coff
                a0 = a0 + r0_v[pl.ds(off, 16)]
                a1 = a1 + r1_v[pl.ds(off, 16)]
            return a0, a1

        a0, a1 = lax.fori_loop(0, S // 8, tb, (zero, zero))
        lenvec = len_v[pl.ds(coff, 16)]
        o0_v[pl.ds(coff, 16)] = a0 * lenvec + b0vec
        o1_v[pl.ds(coff, 16)] = a1 * lenvec + b1vec

    pltpu.sync_copy(o0_v, out0_hbm.at[pl.ds(bbase, BPT)])
    pltpu.sync_copy(o1_v, out1_hbm.at[pl.ds(bbase, BPT)])


def kernel(text, text_lengths, table, W, b):
    textt = _transpose_text(text.astype(jnp.int32))
    inv_len = 1.0 / text_lengths.astype(jnp.float32)
    b16 = jnp.repeat(b.astype(jnp.float32), 16)
    table2 = table.astype(jnp.bfloat16).reshape(VH, 2 * D)
    Wf = W.astype(jnp.float32)
    W2 = jnp.zeros((2 * D, 2 * O), jnp.float32)
    W2 = W2.at[:D, :O].set(Wf).at[D:, O:].set(Wf).astype(jnp.bfloat16)
    e0, e1, q0, q1 = _table_w(table2, W2)
    tw0 = jnp.concatenate([e0, q0])
    tw1 = jnp.concatenate([e1, q1])
    out0, out1 = _sc_pool(textt, inv_len, b16, tw0, tw1)
    return jnp.stack([out0, out1], axis=1)


# speedup vs baseline: 3.2470x; 3.1474x over previous
import functools

import jax
import jax.numpy as jnp
from jax import lax
from jax.experimental import pallas as pl
from jax.experimental.pallas import tpu as pltpu
from jax.experimental.pallas import tpu_sc as plsc

VOCAB = 1000000
D = 64
O = 2
B = 4096
S = 200

NT = 32
BPT = B // NT
TPT = S * BPT
CH = 128
NCH = TPT // CH


def _tw_body(w_ref, t_ref, o0_ref, o1_ref):
    res = lax.dot_general(w_ref[...], t_ref[...], (((1,), (0,)), ((), ())),
                          preferred_element_type=jnp.float32)
    o0_ref[...] = res[0, :]
    o1_ref[...] = res[1, :]


def _table_w(tableT, WT):
    BLK = 16384
    oshape = jax.ShapeDtypeStruct((VOCAB,), jnp.float32)
    ospec = pl.BlockSpec((BLK,), lambda i: (i,))
    return pl.pallas_call(
        _tw_body,
        grid=(pl.cdiv(VOCAB, BLK),),
        in_specs=[
            pl.BlockSpec((O, D), lambda i: (0, 0)),
            pl.BlockSpec((D, BLK), lambda i: (0, i)),
        ],
        out_specs=[ospec, ospec],
        out_shape=[oshape, oshape],
    )(WT, tableT)


def _xp_body(t_ref, o_ref):
    o_ref[...] = t_ref[...].T


def _transpose_text(text):
    return pl.pallas_call(
        _xp_body,
        grid=(NT,),
        in_specs=[pl.BlockSpec((BPT, S), lambda i: (i, 0))],
        out_specs=pl.BlockSpec((S, BPT), lambda i: (i, 0)),
        out_shape=jax.ShapeDtypeStruct((NT * S, BPT), jnp.int32),
    )(text)


@functools.partial(
    pl.kernel,
    out_type=[
        jax.ShapeDtypeStruct((B,), jnp.float32),
        jax.ShapeDtypeStruct((B,), jnp.float32),
    ],
    mesh=plsc.VectorSubcoreMesh(core_axis_name="c", subcore_axis_name="s"),
    scratch_types=[
        pltpu.VMEM((NCH, CH), jnp.int32),
        pltpu.VMEM((TPT,), jnp.float32),
        pltpu.VMEM((TPT,), jnp.float32),
        pltpu.VMEM((BPT,), jnp.float32),
        pltpu.VMEM((32,), jnp.float32),
        pltpu.VMEM((BPT,), jnp.float32),
        pltpu.VMEM((BPT,), jnp.float32),
        pltpu.SemaphoreType.DMA,
    ],
)
def _sc_pool(textt_hbm, len_hbm, b16_hbm, tw0_hbm, tw1_hbm,
             out0_hbm, out1_hbm,
             text_v, r0_v, r1_v, len_v, b_v, o0_v, o1_v, sem):
    wid = lax.axis_index("s") * 2 + lax.axis_index("c")
    bbase = wid * BPT

    pltpu.sync_copy(textt_hbm.at[pl.ds(wid * S, S)], text_v)
    pltpu.sync_copy(len_hbm.at[pl.ds(bbase, BPT)], len_v)
    pltpu.sync_copy(b16_hbm, b_v)

    def chunk_copies(j):
        c0 = pltpu.make_async_copy(tw0_hbm.at[text_v.at[j]],
                                   r0_v.at[pl.ds(j * CH, CH)], sem)
        c1 = pltpu.make_async_copy(tw1_hbm.at[text_v.at[j]],
                                   r1_v.at[pl.ds(j * CH, CH)], sem)
        return c0, c1

    def fire(j, carry):
        c0, c1 = chunk_copies(j)
        c0.start()
        c1.start()

        @pl.when(j > 0)
        def _():
            p0, p1 = chunk_copies(j - 1)
            p0.wait()
            p1.wait()

        return carry

    lax.fori_loop(0, NCH, fire, 0)
    l0, l1 = chunk_copies(NCH - 1)
    l0.wait()
    l1.wait()

    b0vec = b_v[pl.ds(0, 16)]
    b1vec = b_v[pl.ds(16, 16)]
    zero = jnp.zeros((16,), jnp.float32)
    for c in range(BPT // 16):
        coff = c * 16

        def tb(k, accs, coff=coff):
            a0, a1 = accs
            for i in range(8):
                off = (k * 8 + i) * CH + coff
                a0 = a0 + r0_v[pl.ds(off, 16)]
                a1 = a1 + r1_v[pl.ds(off, 16)]
            return a0, a1

        a0, a1 = lax.fori_loop(0, S // 8, tb, (zero, zero))
        lenvec = len_v[pl.ds(coff, 16)]
        o0_v[pl.ds(coff, 16)] = a0 * lenvec + b0vec
        o1_v[pl.ds(coff, 16)] = a1 * lenvec + b1vec

    pltpu.sync_copy(o0_v, out0_hbm.at[pl.ds(bbase, BPT)])
    pltpu.sync_copy(o1_v, out1_hbm.at[pl.ds(bbase, BPT)])


def kernel(text, text_lengths, table, W, b):
    textt = _transpose_text(text.astype(jnp.int32))
    inv_len = 1.0 / text_lengths.astype(jnp.float32)
    b16 = jnp.repeat(b.astype(jnp.float32), 16)
    tw0, tw1 = _table_w(table.T, W.astype(jnp.float32).T)
    out0, out1 = _sc_pool(textt, inv_len, b16, tw0, tw1)
    return jnp.stack([out0, out1], axis=1)


# speedup vs baseline: 3.7933x; 1.1683x over previous
import functools

import jax
import jax.numpy as jnp
from jax import lax
from jax.experimental import pallas as pl
from jax.experimental.pallas import tpu as pltpu
from jax.experimental.pallas import tpu_sc as plsc

VOCAB = 1000000
D = 64
O = 2
B = 4096
S = 200

NT = 32
BPT = B // NT
TPT = S * BPT
CH = 128
NCH = TPT // CH


def _tw_body(w_ref, t_ref, o0_ref, o1_ref):
    res = lax.dot_general(w_ref[...], t_ref[...], (((1,), (0,)), ((), ())),
                          preferred_element_type=jnp.float32)
    o0_ref[...] = res[0, :]
    o1_ref[...] = res[1, :]


def _table_w(tableT, WT):
    BLK = 16384
    oshape = jax.ShapeDtypeStruct((VOCAB,), jnp.float32)
    ospec = pl.BlockSpec((BLK,), lambda i: (i,))
    return pl.pallas_call(
        _tw_body,
        grid=(pl.cdiv(VOCAB, BLK),),
        in_specs=[
            pl.BlockSpec((O, D), lambda i: (0, 0)),
            pl.BlockSpec((D, BLK), lambda i: (0, i)),
        ],
        out_specs=[ospec, ospec],
        out_shape=[oshape, oshape],
    )(WT, tableT)


def _xp_body(t_ref, o_ref):
    o_ref[...] = t_ref[...].T


def _transpose_text(text):
    return pl.pallas_call(
        _xp_body,
        grid=(NT,),
        in_specs=[pl.BlockSpec((BPT, S), lambda i: (i, 0))],
        out_specs=pl.BlockSpec((S, BPT), lambda i: (i, 0)),
        out_shape=jax.ShapeDtypeStruct((NT * S, BPT), jnp.int32),
    )(text)


@functools.partial(
    pl.kernel,
    out_type=[
        jax.ShapeDtypeStruct((B,), jnp.float32),
        jax.ShapeDtypeStruct((B,), jnp.float32),
    ],
    mesh=plsc.VectorSubcoreMesh(core_axis_name="c", subcore_axis_name="s"),
    scratch_types=[
        pltpu.VMEM((NCH, CH), jnp.int32),
        pltpu.VMEM((TPT,), jnp.float32),
        pltpu.VMEM((TPT,), jnp.float32),
        pltpu.VMEM((BPT,), jnp.float32),
        pltpu.VMEM((32,), jnp.float32),
        pltpu.VMEM((BPT,), jnp.float32),
        pltpu.VMEM((BPT,), jnp.float32),
        pltpu.SemaphoreType.DMA,
    ],
)
def _sc_pool(textt_hbm, len_hbm, b16_hbm, tw0_hbm, tw1_hbm,
             out0_hbm, out1_hbm,
             text_v, r0_v, r1_v, len_v, b_v, o0_v, o1_v, sem):
    wid = lax.axis_index("s") * 2 + lax.axis_index("c")
    bbase = wid * BPT

    pltpu.sync_copy(textt_hbm.at[pl.ds(wid * S, S)], text_v)
    pltpu.sync_copy(len_hbm.at[pl.ds(bbase, BPT)], len_v)
    pltpu.sync_copy(b16_hbm, b_v)

    DEPTH = 4

    def chunk_copies(j):
        c0 = pltpu.make_async_copy(tw0_hbm.at[text_v.at[j]],
                                   r0_v.at[pl.ds(j * CH, CH)], sem)
        c1 = pltpu.make_async_copy(tw1_hbm.at[text_v.at[j]],
                                   r1_v.at[pl.ds(j * CH, CH)], sem)
        return c0, c1

    for j in range(DEPTH):
        c0, c1 = chunk_copies(j)
        c0.start()
        c1.start()

    zero = jnp.zeros((16,), jnp.float32)

    def body(j, accs):
        @pl.when(j + DEPTH < NCH)
        def _():
            n0, n1 = chunk_copies(j + DEPTH)
            n0.start()
            n1.start()

        p0, p1 = chunk_copies(j)
        p0.wait()
        p1.wait()
        base = j * CH
        out = []
        for c in range(BPT // 16):
            out.append(accs[2 * c] + r0_v[pl.ds(base + c * 16, 16)])
            out.append(accs[2 * c + 1] + r1_v[pl.ds(base + c * 16, 16)])
        return tuple(out)

    accs = lax.fori_loop(0, NCH, body, (zero,) * 16)

    b0vec = b_v[pl.ds(0, 16)]
    b1vec = b_v[pl.ds(16, 16)]
    for c in range(BPT // 16):
        coff = c * 16
        lenvec = len_v[pl.ds(coff, 16)]
        o0_v[pl.ds(coff, 16)] = accs[2 * c] * lenvec + b0vec
        o1_v[pl.ds(coff, 16)] = accs[2 * c + 1] * lenvec + b1vec

    pltpu.sync_copy(o0_v, out0_hbm.at[pl.ds(bbase, BPT)])
    pltpu.sync_copy(o1_v, out1_hbm.at[pl.ds(bbase, BPT)])


def kernel(text, text_lengths, table, W, b):
    textt = _transpose_text(text.astype(jnp.int32))
    inv_len = 1.0 / text_lengths.astype(jnp.float32)
    b16 = jnp.repeat(b.astype(jnp.float32), 16)
    tw0, tw1 = _table_w(table.T, W.astype(jnp.float32).T)
    out0, out1 = _sc_pool(textt, inv_len, b16, tw0, tw1)
    return jnp.stack([out0, out1], axis=1)


# speedup vs baseline: 4.1569x; 1.0958x over previous
import functools

import jax
import jax.numpy as jnp
from jax import lax
from jax.experimental import pallas as pl
from jax.experimental.pallas import tpu as pltpu
from jax.experimental.pallas import tpu_sc as plsc

VOCAB = 1000000
D = 64
O = 2
B = 4096
S = 200

NT = 32
BPT = B // NT
TPT = S * BPT
CH = 128
NCH = TPT // CH


def _tw_body(w_ref, t_ref, o0_ref, o1_ref):
    res = lax.dot_general(w_ref[...], t_ref[...], (((1,), (0,)), ((), ())),
                          preferred_element_type=jnp.float32)
    o0_ref[...] = res[0, :]
    o1_ref[...] = res[1, :]


def _table_w(tableT, WT):
    BLK = 32768
    oshape = jax.ShapeDtypeStruct((VOCAB,), jnp.float32)
    ospec = pl.BlockSpec((BLK,), lambda i: (i,))
    return pl.pallas_call(
        _tw_body,
        grid=(pl.cdiv(VOCAB, BLK),),
        in_specs=[
            pl.BlockSpec((O, D), lambda i: (0, 0)),
            pl.BlockSpec((D, BLK), lambda i: (0, i)),
        ],
        out_specs=[ospec, ospec],
        out_shape=[oshape, oshape],
    )(WT, tableT)


def _xp_body(t_ref, o_ref):
    o_ref[...] = t_ref[...].T


def _transpose_text(text):
    return pl.pallas_call(
        _xp_body,
        grid=(NT,),
        in_specs=[pl.BlockSpec((BPT, S), lambda i: (i, 0))],
        out_specs=pl.BlockSpec((S, BPT), lambda i: (i, 0)),
        out_shape=jax.ShapeDtypeStruct((NT * S, BPT), jnp.int32),
    )(text)


@functools.partial(
    pl.kernel,
    out_type=[
        jax.ShapeDtypeStruct((B,), jnp.float32),
        jax.ShapeDtypeStruct((B,), jnp.float32),
    ],
    mesh=plsc.VectorSubcoreMesh(core_axis_name="c", subcore_axis_name="s"),
    scratch_types=[
        pltpu.VMEM((NCH, CH), jnp.int32),
        pltpu.VMEM((TPT,), jnp.float32),
        pltpu.VMEM((TPT,), jnp.float32),
        pltpu.VMEM((BPT,), jnp.float32),
        pltpu.VMEM((32,), jnp.float32),
        pltpu.VMEM((BPT,), jnp.float32),
        pltpu.VMEM((BPT,), jnp.float32),
        pltpu.SemaphoreType.DMA,
    ],
)
def _sc_pool(textt_hbm, len_hbm, b16_hbm, tw0_hbm, tw1_hbm,
             out0_hbm, out1_hbm,
             text_v, r0_v, r1_v, len_v, b_v, o0_v, o1_v, sem):
    wid = lax.axis_index("s") * 2 + lax.axis_index("c")
    bbase = wid * BPT

    pltpu.sync_copy(textt_hbm.at[pl.ds(wid * S, S)], text_v)
    pltpu.sync_copy(len_hbm.at[pl.ds(bbase, BPT)], len_v)
    pltpu.sync_copy(b16_hbm, b_v)

    DEPTH = 8

    def chunk_copies(j):
        c0 = pltpu.make_async_copy(tw0_hbm.at[text_v.at[j]],
                                   r0_v.at[pl.ds(j * CH, CH)], sem)
        c1 = pltpu.make_async_copy(tw1_hbm.at[text_v.at[j]],
                                   r1_v.at[pl.ds(j * CH, CH)], sem)
        return c0, c1

    for j in range(DEPTH):
        c0, c1 = chunk_copies(j)
        c0.start()
        c1.start()

    zero = jnp.zeros((16,), jnp.float32)

    def body(j, accs):
        @pl.when(j + DEPTH < NCH)
        def _():
            n0, n1 = chunk_copies(j + DEPTH)
            n0.start()
            n1.start()

        p0, p1 = chunk_copies(j)
        p0.wait()
        p1.wait()
        base = j * CH
        out = []
        for c in range(BPT // 16):
            out.append(accs[2 * c] + r0_v[pl.ds(base + c * 16, 16)])
            out.append(accs[2 * c + 1] + r1_v[pl.ds(base + c * 16, 16)])
        return tuple(out)

    accs = lax.fori_loop(0, NCH, body, (zero,) * 16)

    b0vec = b_v[pl.ds(0, 16)]
    b1vec = b_v[pl.ds(16, 16)]
    for c in range(BPT // 16):
        coff = c * 16
        lenvec = len_v[pl.ds(coff, 16)]
        o0_v[pl.ds(coff, 16)] = accs[2 * c] * lenvec + b0vec
        o1_v[pl.ds(coff, 16)] = accs[2 * c + 1] * lenvec + b1vec

    pltpu.sync_copy(o0_v, out0_hbm.at[pl.ds(bbase, BPT)])
    pltpu.sync_copy(o1_v, out1_hbm.at[pl.ds(bbase, BPT)])


def kernel(text, text_lengths, table, W, b):
    textt = _transpose_text(text.astype(jnp.int32))
    inv_len = 1.0 / text_lengths.astype(jnp.float32)
    b16 = jnp.repeat(b.astype(jnp.float32), 16)
    tw0, tw1 = _table_w(table.T, W.astype(jnp.float32).T)
    out0, out1 = _sc_pool(textt, inv_len, b16, tw0, tw1)
    return jnp.stack([out0, out1], axis=1)


# speedup vs baseline: 4.1779x; 1.0050x over previous
import functools

import jax
import jax.numpy as jnp
from jax import lax
from jax.experimental import pallas as pl
from jax.experimental.pallas import tpu as pltpu
from jax.experimental.pallas import tpu_sc as plsc

VOCAB = 1000000
D = 64
O = 2
B = 4096
S = 200

NT = 32
BPT = B // NT
TPT = S * BPT
CH = 128
NCH = TPT // CH


def _tw_body(w_ref, t_ref, o0_ref, o1_ref):
    res = lax.dot_general(w_ref[...], t_ref[...], (((1,), (0,)), ((), ())),
                          preferred_element_type=jnp.float32)
    o0_ref[...] = res[0, :]
    o1_ref[...] = res[1, :]


def _table_w(tableT, WT):
    BLK = 65536
    oshape = jax.ShapeDtypeStruct((VOCAB,), jnp.float32)
    ospec = pl.BlockSpec((BLK,), lambda i: (i,))
    return pl.pallas_call(
        _tw_body,
        grid=(pl.cdiv(VOCAB, BLK),),
        in_specs=[
            pl.BlockSpec((O, D), lambda i: (0, 0)),
            pl.BlockSpec((D, BLK), lambda i: (0, i)),
        ],
        out_specs=[ospec, ospec],
        out_shape=[oshape, oshape],
    )(WT, tableT)


def _xp_body(t_ref, o_ref):
    o_ref[...] = t_ref[...].T


def _transpose_text(text):
    return pl.pallas_call(
        _xp_body,
        grid=(NT,),
        in_specs=[pl.BlockSpec((BPT, S), lambda i: (i, 0))],
        out_specs=pl.BlockSpec((S, BPT), lambda i: (i, 0)),
        out_shape=jax.ShapeDtypeStruct((NT * S, BPT), jnp.int32),
    )(text)


@functools.partial(
    pl.kernel,
    out_type=[
        jax.ShapeDtypeStruct((B,), jnp.float32),
        jax.ShapeDtypeStruct((B,), jnp.float32),
    ],
    mesh=plsc.VectorSubcoreMesh(core_axis_name="c", subcore_axis_name="s"),
    scratch_types=[
        pltpu.VMEM((NCH, CH), jnp.int32),
        pltpu.VMEM((TPT,), jnp.float32),
        pltpu.VMEM((TPT,), jnp.float32),
        pltpu.VMEM((BPT,), jnp.float32),
        pltpu.VMEM((32,), jnp.float32),
        pltpu.VMEM((BPT,), jnp.float32),
        pltpu.VMEM((BPT,), jnp.float32),
        pltpu.SemaphoreType.DMA,
    ],
)
def _sc_pool(textt_hbm, len_hbm, b16_hbm, tw0_hbm, tw1_hbm,
             out0_hbm, out1_hbm,
             text_v, r0_v, r1_v, len_v, b_v, o0_v, o1_v, sem):
    wid = lax.axis_index("s") * 2 + lax.axis_index("c")
    bbase = wid * BPT

    pltpu.sync_copy(textt_hbm.at[pl.ds(wid * S, S)], text_v)
    pltpu.sync_copy(len_hbm.at[pl.ds(bbase, BPT)], len_v)
    pltpu.sync_copy(b16_hbm, b_v)

    DEPTH = 16

    def chunk_copies(j):
        c0 = pltpu.make_async_copy(tw0_hbm.at[text_v.at[j]],
                                   r0_v.at[pl.ds(j * CH, CH)], sem)
        c1 = pltpu.make_async_copy(tw1_hbm.at[text_v.at[j]],
                                   r1_v.at[pl.ds(j * CH, CH)], sem)
        return c0, c1

    for j in range(DEPTH):
        c0, c1 = chunk_copies(j)
        c0.start()
        c1.start()

    zero = jnp.zeros((16,), jnp.float32)

    def body(j, accs):
        @pl.when(j + DEPTH < NCH)
        def _():
            n0, n1 = chunk_copies(j + DEPTH)
            n0.start()
            n1.start()

        p0, p1 = chunk_copies(j)
        p0.wait()
        p1.wait()
        base = j * CH
        out = []
        for c in range(BPT // 16):
            out.append(accs[2 * c] + r0_v[pl.ds(base + c * 16, 16)])
            out.append(accs[2 * c + 1] + r1_v[pl.ds(base + c * 16, 16)])
        return tuple(out)

    accs = lax.fori_loop(0, NCH, body, (zero,) * 16)

    b0vec = b_v[pl.ds(0, 16)]
    b1vec = b_v[pl.ds(16, 16)]
    for c in range(BPT // 16):
        coff = c * 16
        lenvec = len_v[pl.ds(coff, 16)]
        o0_v[pl.ds(coff, 16)] = accs[2 * c] * lenvec + b0vec
        o1_v[pl.ds(coff, 16)] = accs[2 * c + 1] * lenvec + b1vec

    pltpu.sync_copy(o0_v, out0_hbm.at[pl.ds(bbase, BPT)])
    pltpu.sync_copy(o1_v, out1_hbm.at[pl.ds(bbase, BPT)])


def kernel(text, text_lengths, table, W, b):
    textt = _transpose_text(text.astype(jnp.int32))
    inv_len = 1.0 / text_lengths.astype(jnp.float32)
    b16 = jnp.repeat(b.astype(jnp.float32), 16)
    tw0, tw1 = _table_w(table.T, W.astype(jnp.float32).T)
    out0, out1 = _sc_pool(textt, inv_len, b16, tw0, tw1)
    return jnp.stack([out0, out1], axis=1)
